# baseline (speedup 1.0000x reference)
import functools

import jax
import jax.numpy as jnp
from jax import lax
from jax.experimental import pallas as pl
from jax.experimental.pallas import tpu as pltpu
from jax.experimental.pallas import tpu_sc as plsc

B = 1024
S = 512
DIM = 128
N = B * S
MAX_LEN = 512
N_SEG = 2
NW = 32
PER_W = N // NW
CH = 128
NCH = PER_W // CH
LANES = 16
CGRP = DIM // LANES


def _body(seg_hbm, pos_hbm, tok_hbm, segm_hbm, posm_hbm, tokm_hbm, out_hbm,
          idsA, idsB, comb,
          tokA, tokB, pos_vmem, seg_vmem, pos_stage,
          semIA, semIB, semGA, semGB, semOA, semOB):
  nc = lax.axis_size("c")
  sid = lax.axis_index("s")
  wid = sid * nc + lax.axis_index("c")
  base = wid * PER_W

  pltpu.sync_copy(segm_hbm, seg_vmem)

  seg0 = [seg_vmem[pl.ds(cc * LANES, LANES)] for cc in range(CGRP)]
  d01 = [seg_vmem[pl.ds(DIM + cc * LANES, LANES)] - seg0[cc]
         for cc in range(CGRP)]

  STG = 128
  for k in range(MAX_LEN // STG):
    pltpu.sync_copy(posm_hbm.at[pl.ds(k * STG * DIM, STG * DIM)], pos_stage)

    @plsc.parallel_loop(0, STG)
    def _prow(p):
      for g in range(CGRP // 2):
        a = pos_stage[pl.ds(p * DIM + 2 * g * LANES, LANES)] + seg0[2 * g]
        b = (pos_stage[pl.ds(p * DIM + (2 * g + 1) * LANES, LANES)]
             + seg0[2 * g + 1])
        packed = plsc.pack(a, b, format=plsc.PackFormat.INTERLEAVED)
        pos_vmem[pl.ds((k * STG + p) * (DIM // 2) + g * LANES, LANES)] = (
            plsc.bitcast(packed, jnp.int32))

  def start_ids(c, ids, semI):
    off = base + c * CH
    pltpu.async_copy(seg_hbm.at[pl.ds(off, CH)], ids.at[pl.ds(0, CH)], semI)
    pltpu.async_copy(pos_hbm.at[pl.ds(off, CH)], ids.at[pl.ds(CH, CH)], semI)
    pltpu.async_copy(tok_hbm.at[pl.ds(off, CH)], ids.at[pl.ds(2 * CH, CH)],
                     semI)

  def wait_ids(ids, semI):
    pltpu.make_async_copy(tok_hbm.at[pl.ds(0, 3 * CH)], ids, semI).wait()

  def start_gather(tI, tokR, semG):
    pltpu.async_copy(tokm_hbm.at[tI], tokR, semG)

  def wait_gather(tokR, semG):
    pltpu.make_async_copy(tokm_hbm.at[pl.ds(0, CH)], tokR, semG).wait()

  iota16 = lax.iota(jnp.int32, LANES)
  zeros16 = jnp.zeros((LANES,), jnp.int32)

  def pack_ids(sV, pV):
    @plsc.parallel_loop(0, CH // LANES)
    def _cg(q):
      sl = pl.ds(q * LANES, LANES)
      comb[sl] = (pV[sl] << 6) | (sV[sl] << 22)

  def add_rows(lo, hi, tokR):
    @plsc.parallel_loop(lo, hi, unroll=4)
    def _row(r):
      cb = plsc.load_gather(comb, [zeros16 + r])
      sf = (cb >> 22).astype(jnp.float32)
      pbase = (cb & 0x3FFFFF) + iota16
      for g in range(CGRP // 2):
        words = plsc.load_gather(pos_vmem, [pbase + g * LANES])
        pair = plsc.bitcast(words, jnp.bfloat16)
        a, b = plsc.unpack(pair, format=plsc.PackFormat.INTERLEAVED)
        plsc.addupdate(tokR.at[r, pl.ds(2 * g * LANES, LANES)],
                       a + sf * d01[2 * g])
        plsc.addupdate(tokR.at[r, pl.ds((2 * g + 1) * LANES, LANES)],
                       b + sf * d01[2 * g + 1])

  def write_half(c, tokR, h, semO):
    off = base + c * CH + h * (CH // 2)
    pltpu.async_copy(tokR.at[pl.ds(h * (CH // 2), CH // 2)],
                     out_hbm.at[pl.ds(off, CH // 2)], semO)

  def wait_out(tokR, semO):
    pltpu.make_async_copy(tokR, out_hbm.at[pl.ds(base, CH)], semO).wait()

  bufsA = (idsA, tokA, semIA, semGA, semOA)
  bufsB = (idsB, tokB, semIB, semGB, semOB)

  def process(c, cur, nxt):
    ids, tokR, semI, semG, semO = cur
    ids2, tokR2, semI2, semG2, semO2 = nxt
    wait_ids(ids, semI)
    pl.when(c >= 2)(lambda: wait_out(tokR, semO))
    start_gather(ids.at[pl.ds(2 * CH, CH)], tokR, semG)

    def finish_prev():
      wait_gather(tokR2, semG2)
      pack_ids(ids2.at[pl.ds(0, CH)], ids2.at[pl.ds(CH, CH)])
      add_rows(0, CH // 2, tokR2)
      write_half(c - 1, tokR2, 0, semO2)
      add_rows(CH // 2, CH, tokR2)
      write_half(c - 1, tokR2, 1, semO2)

    pl.when(c >= 1)(finish_prev)
    pl.when(c + 1 < NCH)(lambda: start_ids(c + 1, ids2, semI2))

  start_ids(0, idsA, semIA)

  def pair(j, carry):
    process(2 * j, bufsA, bufsB)
    process(2 * j + 1, bufsB, bufsA)
    return carry

  lax.fori_loop(0, NCH // 2, pair, None)

  wait_gather(tokB, semGB)
  pack_ids(idsB.at[pl.ds(0, CH)], idsB.at[pl.ds(CH, CH)])
  add_rows(0, CH, tokB)
  write_half(NCH - 1, tokB, 0, semOB)
  write_half(NCH - 1, tokB, 1, semOB)
  wait_out(tokA, semOA)
  wait_out(tokB, semOB)


def kernel(segment_ids, pos_ids, token_ids, segment_matrix, pos_matrix,
           token_matrix):
  seg = segment_ids.reshape(N)
  pos = pos_ids.reshape(N)
  tok = token_ids.reshape(N)
  segm_flat = segment_matrix.reshape(N_SEG * DIM)
  posm_flat = pos_matrix.reshape(MAX_LEN * DIM)
  mesh = plsc.VectorSubcoreMesh(core_axis_name="c", subcore_axis_name="s")
  run = pl.kernel(
      _body,
      out_type=jax.ShapeDtypeStruct((N, DIM), jnp.float32),
      mesh=mesh,
      compiler_params=pltpu.CompilerParams(needs_layout_passes=False),
      scratch_types=[
          pltpu.VMEM((3 * CH,), jnp.int32),
          pltpu.VMEM((3 * CH,), jnp.int32),
          pltpu.VMEM((CH,), jnp.int32),
          pltpu.VMEM((CH, DIM), jnp.float32),
          pltpu.VMEM((CH, DIM), jnp.float32),
          pltpu.VMEM((MAX_LEN * DIM // 2,), jnp.int32),
          pltpu.VMEM((N_SEG * DIM,), jnp.float32),
          pltpu.VMEM((128 * DIM,), jnp.float32),
          pltpu.SemaphoreType.DMA,
          pltpu.SemaphoreType.DMA,
          pltpu.SemaphoreType.DMA,
          pltpu.SemaphoreType.DMA,
          pltpu.SemaphoreType.DMA,
          pltpu.SemaphoreType.DMA,
      ],
  )
  out = run(seg, pos, tok, segm_flat, posm_flat, token_matrix)
  return out.reshape(B, S, DIM)


# speedup vs baseline: 1.0110x; 1.0110x over previous
import functools

import jax
import jax.numpy as jnp
from jax import lax
from jax.experimental import pallas as pl
from jax.experimental.pallas import tpu as pltpu
from jax.experimental.pallas import tpu_sc as plsc

B = 1024
S = 512
DIM = 128
N = B * S
MAX_LEN = 512
N_SEG = 2
NW = 32
PER_W = N // NW
CH = 128
NCH = PER_W // CH
LANES = 16
CGRP = DIM // LANES


def _body(seg_hbm, pos_hbm, tok_hbm, segm_hbm, posm_hbm, tokm_hbm, out_hbm,
          idsA, idsB,
          tokA, tokB, pos_vmem, seg_vmem, pos_stage,
          semIA, semIB, semGA, semGB, semOA, semOB):
  nc = lax.axis_size("c")
  sid = lax.axis_index("s")
  wid = sid * nc + lax.axis_index("c")
  base = wid * PER_W

  pltpu.sync_copy(segm_hbm, seg_vmem)

  seg0 = [seg_vmem[pl.ds(cc * LANES, LANES)] for cc in range(CGRP)]
  d01 = [seg_vmem[pl.ds(DIM + cc * LANES, LANES)] - seg0[cc]
         for cc in range(CGRP)]

  STG = 128
  for k in range(MAX_LEN // STG):
    pltpu.sync_copy(posm_hbm.at[pl.ds(k * STG * DIM, STG * DIM)], pos_stage)

    @plsc.parallel_loop(0, STG)
    def _prow(p):
      for g in range(CGRP // 2):
        a = pos_stage[pl.ds(p * DIM + 2 * g * LANES, LANES)] + seg0[2 * g]
        b = (pos_stage[pl.ds(p * DIM + (2 * g + 1) * LANES, LANES)]
             + seg0[2 * g + 1])
        packed = plsc.pack(a, b, format=plsc.PackFormat.INTERLEAVED)
        pos_vmem[pl.ds((k * STG + p) * (DIM // 2) + g * LANES, LANES)] = (
            plsc.bitcast(packed, jnp.int32))

  def start_ids(c, ids, semI):
    off = base + c * CH
    pltpu.async_copy(seg_hbm.at[pl.ds(off, CH)], ids.at[pl.ds(0, CH)], semI)
    pltpu.async_copy(pos_hbm.at[pl.ds(off, CH)], ids.at[pl.ds(CH, CH)], semI)
    pltpu.async_copy(tok_hbm.at[pl.ds(off, CH)], ids.at[pl.ds(2 * CH, CH)],
                     semI)

  def wait_ids(ids, semI):
    pltpu.make_async_copy(tok_hbm.at[pl.ds(0, 3 * CH)], ids, semI).wait()

  def start_gather(tI, tokR, semG):
    pltpu.async_copy(tokm_hbm.at[tI], tokR, semG)

  def wait_gather(tokR, semG):
    pltpu.make_async_copy(tokm_hbm.at[pl.ds(0, CH)], tokR, semG).wait()

  iota16 = lax.iota(jnp.int32, LANES)
  zeros16 = jnp.zeros((LANES,), jnp.int32)

  def add_chunk(sV, pV, tokR):
    @plsc.parallel_loop(0, CH, unroll=8)
    def _row(r):
      pr = plsc.load_gather(pV, [zeros16 + r])
      sr = plsc.load_gather(sV, [zeros16 + r])
      sf = sr.astype(jnp.float32)
      pbase = pr * (DIM // 2) + iota16
      for g in range(CGRP // 2):
        words = plsc.load_gather(pos_vmem, [pbase + g * LANES])
        pair = plsc.bitcast(words, jnp.bfloat16)
        a, b = plsc.unpack(pair, format=plsc.PackFormat.INTERLEAVED)
        plsc.addupdate(tokR.at[r, pl.ds(2 * g * LANES, LANES)],
                       a + sf * d01[2 * g])
        plsc.addupdate(tokR.at[r, pl.ds((2 * g + 1) * LANES, LANES)],
                       b + sf * d01[2 * g + 1])

  def write_out(c, tokR, semO):
    off = base + c * CH
    pltpu.async_copy(tokR, out_hbm.at[pl.ds(off, CH)], semO)

  def wait_out(tokR, semO):
    pltpu.make_async_copy(tokR, out_hbm.at[pl.ds(base, CH)], semO).wait()

  bufsA = (idsA, tokA, semIA, semGA, semOA)
  bufsB = (idsB, tokB, semIB, semGB, semOB)

  def process(c, cur, nxt):
    ids, tokR, semI, semG, semO = cur
    ids2, tokR2, semI2, semG2, semO2 = nxt
    wait_ids(ids, semI)
    pl.when(c >= 2)(lambda: wait_out(tokR, semO))
    start_gather(ids.at[pl.ds(2 * CH, CH)], tokR, semG)

    def finish_prev():
      wait_gather(tokR2, semG2)
      add_chunk(ids2.at[pl.ds(0, CH)], ids2.at[pl.ds(CH, CH)], tokR2)
      write_out(c - 1, tokR2, semO2)

    pl.when(c >= 1)(finish_prev)
    pl.when(c + 1 < NCH)(lambda: start_ids(c + 1, ids2, semI2))

  start_ids(0, idsA, semIA)

  def pair(j, carry):
    process(2 * j, bufsA, bufsB)
    process(2 * j + 1, bufsB, bufsA)
    return carry

  lax.fori_loop(0, NCH // 2, pair, None)

  wait_gather(tokB, semGB)
  add_chunk(idsB.at[pl.ds(0, CH)], idsB.at[pl.ds(CH, CH)], tokB)
  write_out(NCH - 1, tokB, semOB)
  wait_out(tokA, semOA)
  wait_out(tokB, semOB)


def kernel(segment_ids, pos_ids, token_ids, segment_matrix, pos_matrix,
           token_matrix):
  seg = segment_ids.reshape(N)
  pos = pos_ids.reshape(N)
  tok = token_ids.reshape(N)
  segm_flat = segment_matrix.reshape(N_SEG * DIM)
  posm_flat = pos_matrix.reshape(MAX_LEN * DIM)
  mesh = plsc.VectorSubcoreMesh(core_axis_name="c", subcore_axis_name="s")
  run = pl.kernel(
      _body,
      out_type=jax.ShapeDtypeStruct((N, DIM), jnp.float32),
      mesh=mesh,
      compiler_params=pltpu.CompilerParams(needs_layout_passes=False),
      scratch_types=[
          pltpu.VMEM((3 * CH,), jnp.int32),
          pltpu.VMEM((3 * CH,), jnp.int32),
          pltpu.VMEM((CH, DIM), jnp.float32),
          pltpu.VMEM((CH, DIM), jnp.float32),
          pltpu.VMEM((MAX_LEN * DIM // 2,), jnp.int32),
          pltpu.VMEM((N_SEG * DIM,), jnp.float32),
          pltpu.VMEM((128 * DIM,), jnp.float32),
          pltpu.SemaphoreType.DMA,
          pltpu.SemaphoreType.DMA,
          pltpu.SemaphoreType.DMA,
          pltpu.SemaphoreType.DMA,
          pltpu.SemaphoreType.DMA,
          pltpu.SemaphoreType.DMA,
      ],
  )
  out = run(seg, pos, tok, segm_flat, posm_flat, token_matrix)
  return out.reshape(B, S, DIM)


# speedup vs baseline: 1.0884x; 1.0766x over previous
import functools

import jax
import jax.numpy as jnp
from jax import lax
from jax.experimental import pallas as pl
from jax.experimental.pallas import tpu as pltpu
from jax.experimental.pallas import tpu_sc as plsc

B = 1024
S = 512
DIM = 128
N = B * S
MAX_LEN = 512
N_SEG = 2
NW = 32
PER_W = N // NW
CH = 256
NCH = PER_W // CH
LANES = 16
CGRP = DIM // LANES


def _body(seg_hbm, pos_hbm, tok_hbm, segm_hbm, posm_hbm, tokm_hbm, out_hbm,
          idsA, idsB,
          tokA, tokB, pos_vmem, seg_vmem, pos_stage,
          semIA, semIB, semGA, semGB, semOA, semOB):
  nc = lax.axis_size("c")
  sid = lax.axis_index("s")
  wid = sid * nc + lax.axis_index("c")
  base = wid * PER_W

  pltpu.sync_copy(segm_hbm, seg_vmem)

  seg0 = [seg_vmem[pl.ds(cc * LANES, LANES)] for cc in range(CGRP)]
  d01 = [seg_vmem[pl.ds(DIM + cc * LANES, LANES)] - seg0[cc]
         for cc in range(CGRP)]

  STG = 128
  for k in range(MAX_LEN // STG):
    pltpu.sync_copy(posm_hbm.at[pl.ds(k * STG * DIM, STG * DIM)], pos_stage)

    @plsc.parallel_loop(0, STG)
    def _prow(p):
      for g in range(CGRP // 2):
        a = pos_stage[pl.ds(p * DIM + 2 * g * LANES, LANES)] + seg0[2 * g]
        b = (pos_stage[pl.ds(p * DIM + (2 * g + 1) * LANES, LANES)]
             + seg0[2 * g + 1])
        packed = plsc.pack(a, b, format=plsc.PackFormat.INTERLEAVED)
        pos_vmem[pl.ds((k * STG + p) * (DIM // 2) + g * LANES, LANES)] = (
            plsc.bitcast(packed, jnp.int32))

  def start_ids(c, ids, semI):
    off = base + c * CH
    pltpu.async_copy(seg_hbm.at[pl.ds(off, CH)], ids.at[pl.ds(0, CH)], semI)
    pltpu.async_copy(pos_hbm.at[pl.ds(off, CH)], ids.at[pl.ds(CH, CH)], semI)
    pltpu.async_copy(tok_hbm.at[pl.ds(off, CH)], ids.at[pl.ds(2 * CH, CH)],
                     semI)

  def wait_ids(ids, semI):
    pltpu.make_async_copy(tok_hbm.at[pl.ds(0, 3 * CH)], ids, semI).wait()

  def start_gather(ids, tokR, semG):
    for j in range(CH // 128):
      pltpu.async_copy(tokm_hbm.at[ids.at[pl.ds(2 * CH + j * 128, 128)]],
                       tokR.at[pl.ds(j * 128, 128)], semG)

  def wait_gather(tokR, semG):
    pltpu.make_async_copy(tokm_hbm.at[pl.ds(0, CH)], tokR, semG).wait()

  iota16 = lax.iota(jnp.int32, LANES)
  zeros16 = jnp.zeros((LANES,), jnp.int32)

  def add_chunk(sV, pV, tokR):
    @plsc.parallel_loop(0, CH, unroll=8)
    def _row(r):
      pr = plsc.load_gather(pV, [zeros16 + r])
      sr = plsc.load_gather(sV, [zeros16 + r])
      sf = sr.astype(jnp.float32)
      pbase = pr * (DIM // 2) + iota16
      for g in range(CGRP // 2):
        words = plsc.load_gather(pos_vmem, [pbase + g * LANES])
        pair = plsc.bitcast(words, jnp.bfloat16)
        a, b = plsc.unpack(pair, format=plsc.PackFormat.INTERLEAVED)
        plsc.addupdate(tokR.at[r, pl.ds(2 * g * LANES, LANES)],
                       a + sf * d01[2 * g])
        plsc.addupdate(tokR.at[r, pl.ds((2 * g + 1) * LANES, LANES)],
                       b + sf * d01[2 * g + 1])

  def write_out(c, tokR, semO):
    off = base + c * CH
    pltpu.async_copy(tokR, out_hbm.at[pl.ds(off, CH)], semO)

  def wait_out(tokR, semO):
    pltpu.make_async_copy(tokR, out_hbm.at[pl.ds(base, CH)], semO).wait()

  bufsA = (idsA, tokA, semIA, semGA, semOA)
  bufsB = (idsB, tokB, semIB, semGB, semOB)

  def process(c, cur, nxt):
    ids, tokR, semI, semG, semO = cur
    ids2, tokR2, semI2, semG2, semO2 = nxt
    wait_ids(ids, semI)
    pl.when(c >= 2)(lambda: wait_out(tokR, semO))
    start_gather(ids, tokR, semG)

    def finish_prev():
      wait_gather(tokR2, semG2)
      add_chunk(ids2.at[pl.ds(0, CH)], ids2.at[pl.ds(CH, CH)], tokR2)
      write_out(c - 1, tokR2, semO2)

    pl.when(c >= 1)(finish_prev)
    pl.when(c + 1 < NCH)(lambda: start_ids(c + 1, ids2, semI2))

  start_ids(0, idsA, semIA)

  def pair(j, carry):
    process(2 * j, bufsA, bufsB)
    process(2 * j + 1, bufsB, bufsA)
    return carry

  lax.fori_loop(0, NCH // 2, pair, None)

  wait_gather(tokB, semGB)
  add_chunk(idsB.at[pl.ds(0, CH)], idsB.at[pl.ds(CH, CH)], tokB)
  write_out(NCH - 1, tokB, semOB)
  wait_out(tokA, semOA)
  wait_out(tokB, semOB)


def kernel(segment_ids, pos_ids, token_ids, segment_matrix, pos_matrix,
           token_matrix):
  seg = segment_ids.reshape(N)
  pos = pos_ids.reshape(N)
  tok = token_ids.reshape(N)
  segm_flat = segment_matrix.reshape(N_SEG * DIM)
  posm_flat = pos_matrix.reshape(MAX_LEN * DIM)
  mesh = plsc.VectorSubcoreMesh(core_axis_name="c", subcore_axis_name="s")
  run = pl.kernel(
      _body,
      out_type=jax.ShapeDtypeStruct((N, DIM), jnp.float32),
      mesh=mesh,
      compiler_params=pltpu.CompilerParams(needs_layout_passes=False),
      scratch_types=[
          pltpu.VMEM((3 * CH,), jnp.int32),
          pltpu.VMEM((3 * CH,), jnp.int32),
          pltpu.VMEM((CH, DIM), jnp.float32),
          pltpu.VMEM((CH, DIM), jnp.float32),
          pltpu.VMEM((MAX_LEN * DIM // 2,), jnp.int32),
          pltpu.VMEM((N_SEG * DIM,), jnp.float32),
          pltpu.VMEM((128 * DIM,), jnp.float32),
          pltpu.SemaphoreType.DMA,
          pltpu.SemaphoreType.DMA,
          pltpu.SemaphoreType.DMA,
          pltpu.SemaphoreType.DMA,
          pltpu.SemaphoreType.DMA,
          pltpu.SemaphoreType.DMA,
      ],
  )
  out = run(seg, pos, tok, segm_flat, posm_flat, token_matrix)
  return out.reshape(B, S, DIM)
